# Initial kernel scaffold; baseline (speedup 1.0000x reference)
import jax
import jax.numpy as jnp
from jax.experimental import pallas as pl


def kernel(ids, table):
    raise NotImplementedError("write your pallas kernel here")


# speedup vs baseline: 2.9026x; 2.9026x over previous
import functools
import math

import jax
import jax.numpy as jnp
from jax.experimental import pallas as pl
from jax.experimental.pallas import tpu as pltpu


def _gather_kernel(ids_ref, table_ref, out_ref, idx_smem, sem, *,
                   scale, token_block, unroll):
    copy = pltpu.make_async_copy(ids_ref.at[0, 0], idx_smem, sem)
    copy.start()
    copy.wait()

    num_chunks = token_block // unroll

    def chunk_body(c, carry):
        base = c * unroll
        for u in range(unroll):
            t = base + u
            row = table_ref[idx_smem[t], 0]
            out_ref[t, 0] = row * scale
        return carry

    jax.lax.fori_loop(0, num_chunks, chunk_body, 0)


def kernel(ids, table):
    B, S = ids.shape
    V, D = table.shape
    scale = float(math.sqrt(D))

    n_tok = B * S
    TB = 8192
    UNROLL = 32

    n_pad = ((n_tok + TB - 1) // TB) * TB
    flat_ids = ids.reshape(-1).astype(jnp.int32)
    if n_pad != n_tok:
        flat_ids = jnp.pad(flat_ids, (0, n_pad - n_tok))
    n_steps = n_pad // TB

    ids_3d = flat_ids.reshape(n_steps, 1, TB)
    table_3d = table.reshape(V, 1, D)

    out_flat = pl.pallas_call(
        functools.partial(_gather_kernel, scale=scale,
                          token_block=TB, unroll=UNROLL),
        out_shape=jax.ShapeDtypeStruct((n_pad, 1, D), table.dtype),
        grid=(n_steps,),
        in_specs=[
            pl.BlockSpec((1, 1, TB), lambda i: (i, 0, 0)),
            pl.BlockSpec((V, 1, D), lambda i: (0, 0, 0)),
        ],
        out_specs=pl.BlockSpec((TB, 1, D), lambda i: (i, 0, 0)),
        scratch_shapes=[
            pltpu.SMEM((TB,), jnp.int32),
            pltpu.SemaphoreType.DMA,
        ],
        compiler_params=pltpu.CompilerParams(
            dimension_semantics=("parallel",),
        ),
    )(ids_3d, table_3d)

    return out_flat[:n_tok, 0, :].reshape(B, S, D)
